# Initial kernel scaffold; baseline (speedup 1.0000x reference)
import jax
import jax.numpy as jnp
from jax.experimental import pallas as pl


def kernel(x):
    raise NotImplementedError("write your pallas kernel here")


# speedup vs baseline: 1.0683x; 1.0683x over previous
import functools

import numpy as np
import jax
import jax.numpy as jnp
from jax.experimental import pallas as pl
from jax.experimental.pallas import tpu as pltpu


@functools.lru_cache(maxsize=None)
def _filter_consts(H: int, W: int, rate: float):
    n = np.arange(W)
    ang = 2.0 * np.pi * np.outer(n, n) / W
    Wc = np.exp(-1j * ang)
    Vc = np.exp(+1j * ang) / W
    cy, cx = H // 2, W // 2
    rh, rw = int(rate * cy), int(rate * cx)
    cols = np.arange(W)
    col_keep = (~((cols >= cx - rw) & (cols < cx + rw))).astype(np.float64)
    A = np.real((Wc * col_keep[None, :]) @ Vc).astype(np.float32)
    return A, cy - rh, cy + rh


def _row_filter_body(TM, H, lo, hi, x_ref, a_ref, o_ref):
    x = x_ref[...]
    y = jnp.dot(x.astype(jnp.bfloat16), a_ref[...],
                preferred_element_type=jnp.float32)
    base = pl.program_id(0) * TM
    r = (base + jax.lax.broadcasted_iota(jnp.int32, (TM, 1), 0)) % H
    in_band = jnp.logical_and(r >= lo, r < hi)
    o_ref[...] = jnp.where(in_band, y, x)


def kernel(x, rate: float = 0.95):
    B, C, H, W = x.shape
    A_np, lo, hi = _filter_consts(int(H), int(W), float(rate))
    A = jnp.asarray(A_np, dtype=jnp.bfloat16)

    M = B * C * H
    xf = x.reshape(M, W).astype(jnp.float32)

    TM = 512
    while M % TM != 0:
        TM //= 2

    out = pl.pallas_call(
        functools.partial(_row_filter_body, TM, H, lo, hi),
        out_shape=jax.ShapeDtypeStruct((M, W), jnp.float32),
        grid=(M // TM,),
        in_specs=[
            pl.BlockSpec((TM, W), lambda i: (i, 0)),
            pl.BlockSpec((W, W), lambda i: (0, 0)),
        ],
        out_specs=pl.BlockSpec((TM, W), lambda i: (i, 0)),
        compiler_params=pltpu.CompilerParams(
            dimension_semantics=("parallel",),
            vmem_limit_bytes=64 * 2 ** 20),
    )(xf, A)

    return out.reshape(B, C, H, W)


# speedup vs baseline: 3.3008x; 3.0897x over previous
import functools

import numpy as np
import jax
import jax.numpy as jnp
from jax.experimental import pallas as pl
from jax.experimental.pallas import tpu as pltpu


@functools.lru_cache(maxsize=None)
def _filter_consts(H: int, W: int, rate: float):
    n = np.arange(W)
    ang = 2.0 * np.pi * np.outer(n, n) / W
    Wc = np.exp(-1j * ang)
    Vc = np.exp(+1j * ang) / W
    cy, cx = H // 2, W // 2
    rh, rw = int(rate * cy), int(rate * cx)
    cols = np.arange(W)
    col_keep = (~((cols >= cx - rw) & (cols < cx + rw))).astype(np.float64)
    A = np.real((Wc * col_keep[None, :]) @ Vc).astype(np.float32)
    return A, cy - rh, cy + rh


def _row_filter_body(x_ref, a_ref, m_ref, o_ref):
    x = x_ref[...]
    y = jnp.dot(x.astype(jnp.bfloat16), a_ref[...],
                preferred_element_type=jnp.float32)
    o_ref[...] = x + m_ref[...] * y


def kernel(x, rate: float = 0.95):
    B, C, H, W = x.shape
    A_np, lo, hi = _filter_consts(int(H), int(W), float(rate))
    A = jnp.asarray(A_np - np.eye(W, dtype=np.float32), dtype=jnp.bfloat16)

    M = B * C * H
    xf = x.reshape(M, W).astype(jnp.float32)

    TM = 4096
    while M % TM != 0 or TM % H != 0:
        TM //= 2

    r = np.arange(TM) % H
    mask = jnp.asarray(((r >= lo) & (r < hi)).reshape(TM, 1).astype(np.float32))

    out = pl.pallas_call(
        _row_filter_body,
        out_shape=jax.ShapeDtypeStruct((M, W), jnp.float32),
        grid=(M // TM,),
        in_specs=[
            pl.BlockSpec((TM, W), lambda i: (i, 0)),
            pl.BlockSpec((W, W), lambda i: (0, 0)),
            pl.BlockSpec((TM, 1), lambda i: (0, 0)),
        ],
        out_specs=pl.BlockSpec((TM, W), lambda i: (i, 0)),
        compiler_params=pltpu.CompilerParams(
            dimension_semantics=("parallel",),
            vmem_limit_bytes=64 * 2 ** 20),
    )(xf, A, mask)

    return out.reshape(B, C, H, W)


# speedup vs baseline: 3.6342x; 1.1010x over previous
import functools

import numpy as np
import jax
import jax.numpy as jnp
from jax.experimental import pallas as pl
from jax.experimental.pallas import tpu as pltpu


@functools.lru_cache(maxsize=None)
def _filter_consts(H: int, W: int, rate: float):
    n = np.arange(W)
    ang = 2.0 * np.pi * np.outer(n, n) / W
    Wc = np.exp(-1j * ang)
    Vc = np.exp(+1j * ang) / W
    cy, cx = H // 2, W // 2
    rh, rw = int(rate * cy), int(rate * cx)
    cols = np.arange(W)
    col_keep = (~((cols >= cx - rw) & (cols < cx + rw))).astype(np.float64)
    A = np.real((Wc * col_keep[None, :]) @ Vc).astype(np.float32)
    return A, cy - rh, cy + rh


def _row_filter_body(x_ref, a_ref, m_ref, o_ref):
    x = x_ref[...]
    y = jnp.dot(x.astype(jnp.bfloat16), a_ref[...],
                preferred_element_type=jnp.float32)
    o_ref[...] = x + m_ref[...] * y


def kernel(x, rate: float = 0.95):
    B, C, H, W = x.shape
    A_np, lo, hi = _filter_consts(int(H), int(W), float(rate))
    A = jnp.asarray(A_np - np.eye(W, dtype=np.float32), dtype=jnp.bfloat16)

    M = B * C * H
    xf = x.reshape(M, W).astype(jnp.float32)

    TM = 8192
    while M % TM != 0 or TM % H != 0:
        TM //= 2

    r = np.arange(TM) % H
    mask = jnp.asarray(((r >= lo) & (r < hi)).reshape(TM, 1).astype(np.float32))

    out = pl.pallas_call(
        _row_filter_body,
        out_shape=jax.ShapeDtypeStruct((M, W), jnp.float32),
        grid=(M // TM,),
        in_specs=[
            pl.BlockSpec((TM, W), lambda i: (i, 0)),
            pl.BlockSpec((W, W), lambda i: (0, 0)),
            pl.BlockSpec((TM, 1), lambda i: (0, 0)),
        ],
        out_specs=pl.BlockSpec((TM, W), lambda i: (i, 0)),
        compiler_params=pltpu.CompilerParams(
            dimension_semantics=("parallel",),
            vmem_limit_bytes=64 * 2 ** 20),
    )(xf, A, mask)

    return out.reshape(B, C, H, W)
